# baseline (speedup 1.0000x reference)
import functools

import jax
import jax.numpy as jnp
from jax import lax
from jax.experimental import pallas as pl
from jax.experimental.pallas import tpu as pltpu
from jax.experimental.pallas import tpu_sc as plsc

F32 = jnp.float32

NC = 2
NS = 16
NW = NC * NS
H = 64
NPAD = 10240
BT = 1024

_SC_PARAMS = pltpu.CompilerParams(use_tc_tiling_on_sc=False)


def _fill_tail(idx_ref, start, count, base):
    def body(t, carry):
        idx_ref[pl.ds(start + t * 16, 16)] = base + t * 16 + lax.iota(
            jnp.int32, 16)
        return carry

    lax.fori_loop(0, count // 16, body, 0)


def _zero_rows(rows_ref, nr, F):
    z16 = jnp.zeros((16,), F32)

    def body(r, carry):
        for cc in range(F // 16):
            rows_ref[r, pl.ds(cc * 16, 16)] = z16
        return carry

    lax.fori_loop(0, nr, body, 0)


def _make_deg_kernel(N, E):
    EW = E // NW
    EWP = -(-EW // 128) * 128
    NCH = EWP // 128
    mesh = plsc.VectorSubcoreMesh(core_axis_name="c", subcore_axis_name="s")

    @functools.partial(
        pl.kernel,
        out_type=jax.ShapeDtypeStruct((NC * NPAD,), F32),
        mesh=mesh,
        scratch_types=[
            pltpu.VMEM((EWP,), jnp.int32),
            pltpu.VMEM((128,), F32),
            pltpu.VMEM((2048,), F32),
            pltpu.VMEM_SHARED((NPAD,), F32),
            pltpu.SemaphoreType.DMA,
        ],
        compiler_params=_SC_PARAMS,
    )
    def deg_kernel(ei_hbm, ones_hbm, out_hbm, dst_v, ones_v, zbuf, acc, sem):
        c = lax.axis_index("c")
        s = lax.axis_index("s")
        wid = s * NC + c
        pltpu.sync_copy(ei_hbm.at[1, pl.ds(wid * EW, EW)],
                        dst_v.at[pl.ds(0, EW)])
        pltpu.sync_copy(ones_hbm, ones_v)
        _fill_tail(dst_v, EW, EWP - EW, N)

        @pl.when(s == 0)
        def _():
            def zrow(r, carry):
                zbuf[pl.ds(r * 16, 16)] = jnp.zeros((16,), F32)
                return carry

            lax.fori_loop(0, 128, zrow, 0)
            for t in range(NPAD // 2048):
                pltpu.sync_copy(zbuf, acc.at[pl.ds(t * 2048, 2048)])

        plsc.subcore_barrier()

        def body(j, carry):
            pltpu.sync_copy(ones_v, acc.at[dst_v.at[pl.ds(j * 128, 128)]],
                            add=True)
            return carry

        lax.fori_loop(0, NCH, body, 0)
        plsc.subcore_barrier()

        @pl.when(s == 0)
        def _():
            pltpu.sync_copy(acc, out_hbm.at[pl.ds(c * NPAD, NPAD)])

    return deg_kernel


def _agg_pipeline(table, hbm_tab, src_v, dst_v, rows_a, rows_b, acc,
                  sem_a, sem_b, n_halves):
    def idx_a(j):
        return src_v.at[pl.ds((2 * j) * H, H)]

    def idx_b(j):
        return src_v.at[pl.ds((2 * j + 1) * H, H)]

    pltpu.async_copy(table.at[idx_a(0)], rows_a, sem_a)

    def body(j, carry):
        pltpu.make_async_copy(table.at[idx_a(j)], rows_a, sem_a).wait()
        pltpu.async_copy(hbm_tab.at[idx_b(j)], rows_b, sem_b)
        pltpu.sync_copy(rows_a, acc.at[dst_v.at[pl.ds((2 * j) * H, H)]],
                        add=True)
        pltpu.make_async_copy(hbm_tab.at[idx_b(j)], rows_b, sem_b).wait()

        @pl.when(j + 1 < n_halves // 2)
        def _():
            pltpu.async_copy(table.at[idx_a(j + 1)], rows_a, sem_a)

        pltpu.sync_copy(rows_b, acc.at[dst_v.at[pl.ds((2 * j + 1) * H, H)]],
                        add=True)
        return carry

    lax.fori_loop(0, n_halves // 2, body, 0)


def _make_agg_kernel(N, E, F):
    EW = E // NW
    EWP = -(-EW // 128) * 128
    NROW = NPAD // NS
    TR = NPAD // NS
    mesh = plsc.VectorSubcoreMesh(core_axis_name="c", subcore_axis_name="s")

    @functools.partial(
        pl.kernel,
        out_type=jax.ShapeDtypeStruct((NC * NPAD, F), F32),
        mesh=mesh,
        scratch_types=[
            pltpu.VMEM((EWP,), jnp.int32),
            pltpu.VMEM((EWP,), jnp.int32),
            pltpu.VMEM((H, F), F32),
            pltpu.VMEM((H, F), F32),
            pltpu.VMEM_SHARED((NPAD, F), F32),
            pltpu.VMEM_SHARED((NPAD, F), F32),
            pltpu.SemaphoreType.DMA,
            pltpu.SemaphoreType.DMA,
        ],
        compiler_params=_SC_PARAMS,
    )
    def agg_kernel(g_hbm, ei_hbm, out_hbm,
                   src_v, dst_v, rows_a, rows_b, acc, table, sem_a, sem_b):
        c = lax.axis_index("c")
        s = lax.axis_index("s")
        wid = s * NC + c
        pltpu.sync_copy(g_hbm.at[pl.ds(s * TR, TR)],
                        table.at[pl.ds(s * TR, TR)])
        pltpu.sync_copy(ei_hbm.at[0, pl.ds(wid * EW, EW)],
                        src_v.at[pl.ds(0, EW)])
        pltpu.sync_copy(ei_hbm.at[1, pl.ds(wid * EW, EW)],
                        dst_v.at[pl.ds(0, EW)])
        _fill_tail(src_v, EW, EWP - EW, 0)
        _fill_tail(dst_v, EW, EWP - EW, N)

        _zero_rows(rows_a, H, F)
        for t in range(NROW // H):
            pltpu.sync_copy(rows_a, acc.at[pl.ds(s * NROW + t * H, H)])
        plsc.subcore_barrier()

        _agg_pipeline(table, g_hbm, src_v, dst_v, rows_a, rows_b, acc,
                      sem_a, sem_b, EWP // H)
        plsc.subcore_barrier()
        pltpu.sync_copy(acc.at[pl.ds(s * NROW, NROW)],
                        out_hbm.at[pl.ds(c * NPAD + s * NROW, NROW)])

    return agg_kernel


def _make_agg_split_kernel(N, E, HF):
    EW = E // NW
    EW2 = 2 * EW
    EWP2 = -(-EW2 // 128) * 128
    NROW = NPAD // NS
    TR = NPAD // NS
    mesh = plsc.VectorSubcoreMesh(core_axis_name="c", subcore_axis_name="s")

    @functools.partial(
        pl.kernel,
        out_type=jax.ShapeDtypeStruct((NC * NPAD, HF), F32),
        mesh=mesh,
        scratch_types=[
            pltpu.VMEM((EWP2,), jnp.int32),
            pltpu.VMEM((EWP2,), jnp.int32),
            pltpu.VMEM((H, HF), F32),
            pltpu.VMEM((H, HF), F32),
            pltpu.VMEM_SHARED((NPAD, HF), F32),
            pltpu.VMEM_SHARED((NPAD, HF), F32),
            pltpu.SemaphoreType.DMA,
            pltpu.SemaphoreType.DMA,
        ],
        compiler_params=_SC_PARAMS,
    )
    def agg_kernel(ga_hbm, gb_hbm, ei_hbm, out_hbm,
                   src_v, dst_v, rows_a, rows_b, acc, table, sem_a, sem_b):
        c = lax.axis_index("c")
        s = lax.axis_index("s")

        @pl.when(c == 0)
        def _():
            pltpu.sync_copy(ga_hbm.at[pl.ds(s * TR, TR)],
                            table.at[pl.ds(s * TR, TR)])

        @pl.when(c == 1)
        def _():
            pltpu.sync_copy(gb_hbm.at[pl.ds(s * TR, TR)],
                            table.at[pl.ds(s * TR, TR)])

        pltpu.sync_copy(ei_hbm.at[0, pl.ds(s * EW2, EW2)],
                        src_v.at[pl.ds(0, EW2)])
        pltpu.sync_copy(ei_hbm.at[1, pl.ds(s * EW2, EW2)],
                        dst_v.at[pl.ds(0, EW2)])
        _fill_tail(src_v, EW2, EWP2 - EW2, 0)
        _fill_tail(dst_v, EW2, EWP2 - EW2, N)

        _zero_rows(rows_a, H, HF)
        for t in range(NROW // H):
            pltpu.sync_copy(rows_a, acc.at[pl.ds(s * NROW + t * H, H)])
        plsc.subcore_barrier()

        @pl.when(c == 0)
        def _():
            _agg_pipeline(table, ga_hbm, src_v, dst_v, rows_a, rows_b, acc,
                          sem_a, sem_b, EWP2 // H)

        @pl.when(c == 1)
        def _():
            _agg_pipeline(table, gb_hbm, src_v, dst_v, rows_a, rows_b, acc,
                          sem_a, sem_b, EWP2 // H)

        plsc.subcore_barrier()
        pltpu.sync_copy(acc.at[pl.ds(s * NROW, NROW)],
                        out_hbm.at[pl.ds(c * NPAD + s * NROW, NROW)])

    return agg_kernel


def _g1_body(x_ref, degp_ref, w_ref, ga_ref, gb_ref, dis_ref):
    deg = degp_ref[:, 0:1] + degp_ref[:, 1:2] + 1.0
    dis = lax.rsqrt(deg)
    dis_ref[...] = dis
    g = jnp.dot(x_ref[...] * dis, w_ref[...], preferred_element_type=F32)
    ga_ref[...] = g[:, :64]
    gb_ref[...] = g[:, 64:]


def _layer12_body(aggA_ref, aggB_ref, ga_ref, gb_ref, dis_ref, b_ref, w_ref,
                  o_ref):
    dis = dis_ref[...]
    ha = jnp.maximum((aggA_ref[...] + ga_ref[...]) * dis + b_ref[:, :64], 0.0)
    hb = jnp.maximum((aggB_ref[...] + gb_ref[...]) * dis + b_ref[:, 64:], 0.0)
    o_ref[...] = (jnp.dot(ha * dis, w_ref[:64], preferred_element_type=F32) +
                  jnp.dot(hb * dis, w_ref[64:], preferred_element_type=F32))


def _layer_body(p0_ref, p1_ref, g_ref, dis_ref, b_ref, w_ref, o_ref):
    a = p0_ref[...] + p1_ref[...] + g_ref[...]
    h = jnp.maximum(a * dis_ref[...] + b_ref[...], 0.0)
    o_ref[...] = jnp.dot(h * dis_ref[...], w_ref[...],
                         preferred_element_type=F32)


def _make_final_body(NG, NB, FH):
    def final_body(p0_ref, p1_ref, g_ref, dis_ref, b_ref, batch_ref, wc_ref,
                   bc_ref, o_ref, acc_ref):
        i = pl.program_id(0)

        @pl.when(i == 0)
        def _():
            acc_ref[...] = jnp.zeros_like(acc_ref)

        a = p0_ref[...] + p1_ref[...] + g_ref[...]
        h = jnp.maximum(a * dis_ref[...] + b_ref[...], 0.0)
        b = batch_ref[0, 0, :]
        onehot = (b[None, :] ==
                  lax.broadcasted_iota(jnp.int32, (NG, BT), 0)).astype(F32)
        hx = jnp.concatenate([h, jnp.ones((BT, 1), F32)], axis=1)
        acc_ref[...] += jnp.dot(onehot, hx, preferred_element_type=F32)

        @pl.when(i == NB - 1)
        def _():
            sums = acc_ref[:, :FH]
            cnt = jnp.maximum(acc_ref[:, FH:FH + 1], 1.0)
            pooled = sums / cnt
            logits = jnp.dot(pooled, wc_ref[...], preferred_element_type=F32) \
                + bc_ref[...]
            m = jnp.max(logits, axis=1, keepdims=True)
            lse = jnp.log(jnp.sum(jnp.exp(logits - m), axis=1,
                                  keepdims=True)) + m
            o_ref[...] = logits - lse

    return final_body


def kernel(x, edge_index, batch, W1, b1, W2, b2, W3, b3, Wc, bc):
    N, F_IN = x.shape
    E = edge_index.shape[1]
    NG = 64
    NB = NPAD // BT

    ei = edge_index.astype(jnp.int32)
    xp = jnp.pad(x, ((0, NPAD - N), (0, 0)))
    batchp = jnp.pad(batch.astype(jnp.int32), (0, NPAD - N),
                     constant_values=NG)
    batch3 = batchp.reshape(NB, 1, BT)
    ones_k = jnp.ones((128,), F32)

    degp = _make_deg_kernel(N, E)(ei, ones_k)
    degpT = degp.reshape(NC, NPAD).T

    g1a, g1b, dis = pl.pallas_call(
        _g1_body,
        grid=(NB,),
        in_specs=[pl.BlockSpec((BT, F_IN), lambda i: (i, 0)),
                  pl.BlockSpec((BT, 2), lambda i: (i, 0)),
                  pl.BlockSpec((F_IN, 128), lambda i: (0, 0))],
        out_specs=[pl.BlockSpec((BT, 64), lambda i: (i, 0)),
                   pl.BlockSpec((BT, 64), lambda i: (i, 0)),
                   pl.BlockSpec((BT, 1), lambda i: (i, 0))],
        out_shape=[jax.ShapeDtypeStruct((NPAD, 64), F32),
                   jax.ShapeDtypeStruct((NPAD, 64), F32),
                   jax.ShapeDtypeStruct((NPAD, 1), F32)],
    )(xp, degpT, W1)

    agg1 = _make_agg_split_kernel(N, E, 64)(g1a, g1b, ei)
    g2 = pl.pallas_call(
        _layer12_body,
        grid=(NB,),
        in_specs=[pl.BlockSpec((BT, 64), lambda i: (i, 0)),
                  pl.BlockSpec((BT, 64), lambda i: (NB + i, 0)),
                  pl.BlockSpec((BT, 64), lambda i: (i, 0)),
                  pl.BlockSpec((BT, 64), lambda i: (i, 0)),
                  pl.BlockSpec((BT, 1), lambda i: (i, 0)),
                  pl.BlockSpec((1, 128), lambda i: (0, 0)),
                  pl.BlockSpec((128, 64), lambda i: (0, 0))],
        out_specs=pl.BlockSpec((BT, 64), lambda i: (i, 0)),
        out_shape=jax.ShapeDtypeStruct((NPAD, 64), F32),
    )(agg1, agg1, g1a, g1b, dis, b1.reshape(1, -1), W2)

    agg2 = _make_agg_kernel(N, E, 64)(g2, ei)
    g3 = pl.pallas_call(
        _layer_body,
        grid=(NB,),
        in_specs=[pl.BlockSpec((BT, 64), lambda i: (i, 0)),
                  pl.BlockSpec((BT, 64), lambda i: (NB + i, 0)),
                  pl.BlockSpec((BT, 64), lambda i: (i, 0)),
                  pl.BlockSpec((BT, 1), lambda i: (i, 0)),
                  pl.BlockSpec((1, 64), lambda i: (0, 0)),
                  pl.BlockSpec((64, 32), lambda i: (0, 0))],
        out_specs=pl.BlockSpec((BT, 32), lambda i: (i, 0)),
        out_shape=jax.ShapeDtypeStruct((NPAD, 32), F32),
    )(agg2, agg2, g2, dis, b2.reshape(1, -1), W3)

    agg3 = _make_agg_kernel(N, E, 32)(g3, ei)

    out = pl.pallas_call(
        _make_final_body(NG, NB, 32),
        grid=(NB,),
        in_specs=[pl.BlockSpec((BT, 32), lambda i: (i, 0)),
                  pl.BlockSpec((BT, 32), lambda i: (NB + i, 0)),
                  pl.BlockSpec((BT, 32), lambda i: (i, 0)),
                  pl.BlockSpec((BT, 1), lambda i: (i, 0)),
                  pl.BlockSpec((1, 32), lambda i: (0, 0)),
                  pl.BlockSpec((1, 1, BT), lambda i: (i, 0, 0)),
                  pl.BlockSpec((32, 10), lambda i: (0, 0)),
                  pl.BlockSpec((1, 10), lambda i: (0, 0))],
        out_specs=pl.BlockSpec((NG, 10), lambda i: (0, 0)),
        out_shape=jax.ShapeDtypeStruct((NG, 10), F32),
        scratch_shapes=[pltpu.VMEM((NG, 33), F32)],
    )(agg3, agg3, g3, dis, b3.reshape(1, -1), batch3, Wc, bc.reshape(1, -1))

    return out


# speedup vs baseline: 1.2454x; 1.2454x over previous
import functools

import jax
import jax.numpy as jnp
from jax import lax
from jax.experimental import pallas as pl
from jax.experimental.pallas import tpu as pltpu
from jax.experimental.pallas import tpu_sc as plsc

F32 = jnp.float32

NC = 2
NS = 16
NW = NC * NS
H = 64
NPAD = 10240
BT = 1024

_SC_PARAMS = pltpu.CompilerParams(use_tc_tiling_on_sc=False)


def _fill_tail(idx_ref, start, count, base):
    def body(t, carry):
        idx_ref[pl.ds(start + t * 16, 16)] = base + t * 16 + lax.iota(
            jnp.int32, 16)
        return carry

    lax.fori_loop(0, count // 16, body, 0)


def _zero_rows(rows_ref, nr, F):
    z16 = jnp.zeros((16,), F32)

    def body(r, carry):
        for cc in range(F // 16):
            rows_ref[r, pl.ds(cc * 16, 16)] = z16
        return carry

    lax.fori_loop(0, nr, body, 0)


def _make_deg_kernel(N, E):
    EW = E // NW
    EWP = -(-EW // 128) * 128
    NCH = EWP // 128
    mesh = plsc.VectorSubcoreMesh(core_axis_name="c", subcore_axis_name="s")

    @functools.partial(
        pl.kernel,
        out_type=jax.ShapeDtypeStruct((NC * NPAD,), F32),
        mesh=mesh,
        scratch_types=[
            pltpu.VMEM((EWP,), jnp.int32),
            pltpu.VMEM((128,), F32),
            pltpu.VMEM((2048,), F32),
            pltpu.VMEM_SHARED((NPAD,), F32),
            pltpu.SemaphoreType.DMA,
        ],
        compiler_params=_SC_PARAMS,
    )
    def deg_kernel(ei_hbm, ones_hbm, out_hbm, dst_v, ones_v, zbuf, acc, sem):
        c = lax.axis_index("c")
        s = lax.axis_index("s")
        wid = s * NC + c
        pltpu.sync_copy(ei_hbm.at[1, pl.ds(wid * EW, EW)],
                        dst_v.at[pl.ds(0, EW)])
        pltpu.sync_copy(ones_hbm, ones_v)
        _fill_tail(dst_v, EW, EWP - EW, N)

        @pl.when(s == 0)
        def _():
            def zrow(r, carry):
                zbuf[pl.ds(r * 16, 16)] = jnp.zeros((16,), F32)
                return carry

            lax.fori_loop(0, 128, zrow, 0)
            for t in range(NPAD // 2048):
                pltpu.sync_copy(zbuf, acc.at[pl.ds(t * 2048, 2048)])

        plsc.subcore_barrier()

        def body(j, carry):
            pltpu.sync_copy(ones_v, acc.at[dst_v.at[pl.ds(j * 128, 128)]],
                            add=True)
            return carry

        lax.fori_loop(0, NCH, body, 0)
        plsc.subcore_barrier()

        @pl.when(s == 0)
        def _():
            pltpu.sync_copy(acc, out_hbm.at[pl.ds(c * NPAD, NPAD)])

    return deg_kernel


def _agg_pipeline(table, hbm_tab, src_v, dst_v, rows_a, rows_b, acc,
                  sem_a, sem_b, n_halves):
    def idx_a(j):
        return src_v.at[pl.ds((2 * j) * H, H)]

    def idx_b(j):
        return src_v.at[pl.ds((2 * j + 1) * H, H)]

    pltpu.async_copy(table.at[idx_a(0)], rows_a, sem_a)

    def body(j, carry):
        pltpu.make_async_copy(table.at[idx_a(j)], rows_a, sem_a).wait()
        pltpu.async_copy(table.at[idx_b(j)], rows_b, sem_b)
        pltpu.sync_copy(rows_a, acc.at[dst_v.at[pl.ds((2 * j) * H, H)]],
                        add=True)
        pltpu.make_async_copy(table.at[idx_b(j)], rows_b, sem_b).wait()

        @pl.when(j + 1 < n_halves // 2)
        def _():
            pltpu.async_copy(table.at[idx_a(j + 1)], rows_a, sem_a)

        pltpu.sync_copy(rows_b, acc.at[dst_v.at[pl.ds((2 * j + 1) * H, H)]],
                        add=True)
        return carry

    lax.fori_loop(0, n_halves // 2, body, 0)


def _make_agg_kernel(N, E, F):
    EW = E // NW
    EWP = -(-EW // 128) * 128
    NROW = NPAD // NS
    TR = NPAD // NS
    mesh = plsc.VectorSubcoreMesh(core_axis_name="c", subcore_axis_name="s")

    @functools.partial(
        pl.kernel,
        out_type=jax.ShapeDtypeStruct((NC * NPAD, F), F32),
        mesh=mesh,
        scratch_types=[
            pltpu.VMEM((EWP,), jnp.int32),
            pltpu.VMEM((EWP,), jnp.int32),
            pltpu.VMEM((H, F), F32),
            pltpu.VMEM((H, F), F32),
            pltpu.VMEM_SHARED((NPAD, F), F32),
            pltpu.VMEM_SHARED((NPAD, F), F32),
            pltpu.SemaphoreType.DMA,
            pltpu.SemaphoreType.DMA,
        ],
        compiler_params=_SC_PARAMS,
    )
    def agg_kernel(g_hbm, ei_hbm, out_hbm,
                   src_v, dst_v, rows_a, rows_b, acc, table, sem_a, sem_b):
        c = lax.axis_index("c")
        s = lax.axis_index("s")
        wid = s * NC + c
        pltpu.sync_copy(g_hbm.at[pl.ds(s * TR, TR)],
                        table.at[pl.ds(s * TR, TR)])
        pltpu.sync_copy(ei_hbm.at[0, pl.ds(wid * EW, EW)],
                        src_v.at[pl.ds(0, EW)])
        pltpu.sync_copy(ei_hbm.at[1, pl.ds(wid * EW, EW)],
                        dst_v.at[pl.ds(0, EW)])
        _fill_tail(src_v, EW, EWP - EW, 0)
        _fill_tail(dst_v, EW, EWP - EW, N)

        _zero_rows(rows_a, H, F)
        for t in range(NROW // H):
            pltpu.sync_copy(rows_a, acc.at[pl.ds(s * NROW + t * H, H)])
        plsc.subcore_barrier()

        _agg_pipeline(table, g_hbm, src_v, dst_v, rows_a, rows_b, acc,
                      sem_a, sem_b, EWP // H)
        plsc.subcore_barrier()
        pltpu.sync_copy(acc.at[pl.ds(s * NROW, NROW)],
                        out_hbm.at[pl.ds(c * NPAD + s * NROW, NROW)])

    return agg_kernel


def _make_agg_split_kernel(N, E, HF):
    EW = E // NW
    EW2 = 2 * EW
    EWP2 = -(-EW2 // 128) * 128
    NROW = NPAD // NS
    TR = NPAD // NS
    mesh = plsc.VectorSubcoreMesh(core_axis_name="c", subcore_axis_name="s")

    @functools.partial(
        pl.kernel,
        out_type=jax.ShapeDtypeStruct((NC * NPAD, HF), F32),
        mesh=mesh,
        scratch_types=[
            pltpu.VMEM((EWP2,), jnp.int32),
            pltpu.VMEM((EWP2,), jnp.int32),
            pltpu.VMEM((H, HF), F32),
            pltpu.VMEM((H, HF), F32),
            pltpu.VMEM_SHARED((NPAD, HF), F32),
            pltpu.VMEM_SHARED((NPAD, HF), F32),
            pltpu.SemaphoreType.DMA,
            pltpu.SemaphoreType.DMA,
        ],
        compiler_params=_SC_PARAMS,
    )
    def agg_kernel(ga_hbm, gb_hbm, ei_hbm, out_hbm,
                   src_v, dst_v, rows_a, rows_b, acc, table, sem_a, sem_b):
        c = lax.axis_index("c")
        s = lax.axis_index("s")

        @pl.when(c == 0)
        def _():
            pltpu.sync_copy(ga_hbm.at[pl.ds(s * TR, TR)],
                            table.at[pl.ds(s * TR, TR)])

        @pl.when(c == 1)
        def _():
            pltpu.sync_copy(gb_hbm.at[pl.ds(s * TR, TR)],
                            table.at[pl.ds(s * TR, TR)])

        pltpu.sync_copy(ei_hbm.at[0, pl.ds(s * EW2, EW2)],
                        src_v.at[pl.ds(0, EW2)])
        pltpu.sync_copy(ei_hbm.at[1, pl.ds(s * EW2, EW2)],
                        dst_v.at[pl.ds(0, EW2)])
        _fill_tail(src_v, EW2, EWP2 - EW2, 0)
        _fill_tail(dst_v, EW2, EWP2 - EW2, N)

        _zero_rows(rows_a, H, HF)
        for t in range(NROW // H):
            pltpu.sync_copy(rows_a, acc.at[pl.ds(s * NROW + t * H, H)])
        plsc.subcore_barrier()

        @pl.when(c == 0)
        def _():
            _agg_pipeline(table, ga_hbm, src_v, dst_v, rows_a, rows_b, acc,
                          sem_a, sem_b, EWP2 // H)

        @pl.when(c == 1)
        def _():
            _agg_pipeline(table, gb_hbm, src_v, dst_v, rows_a, rows_b, acc,
                          sem_a, sem_b, EWP2 // H)

        plsc.subcore_barrier()
        pltpu.sync_copy(acc.at[pl.ds(s * NROW, NROW)],
                        out_hbm.at[pl.ds(c * NPAD + s * NROW, NROW)])

    return agg_kernel


def _g1_body(x_ref, degp_ref, w_ref, ga_ref, gb_ref, dis_ref):
    deg = degp_ref[:, 0:1] + degp_ref[:, 1:2] + 1.0
    dis = lax.rsqrt(deg)
    dis_ref[...] = dis
    g = jnp.dot(x_ref[...] * dis, w_ref[...], preferred_element_type=F32)
    ga_ref[...] = g[:, :64]
    gb_ref[...] = g[:, 64:]


def _layer12_body(aggA_ref, aggB_ref, ga_ref, gb_ref, dis_ref, b_ref, w_ref,
                  o_ref):
    dis = dis_ref[...]
    ha = jnp.maximum((aggA_ref[...] + ga_ref[...]) * dis + b_ref[:, :64], 0.0)
    hb = jnp.maximum((aggB_ref[...] + gb_ref[...]) * dis + b_ref[:, 64:], 0.0)
    o_ref[...] = (jnp.dot(ha * dis, w_ref[:64], preferred_element_type=F32) +
                  jnp.dot(hb * dis, w_ref[64:], preferred_element_type=F32))


def _layer_body(p0_ref, p1_ref, g_ref, dis_ref, b_ref, w_ref, o_ref):
    a = p0_ref[...] + p1_ref[...] + g_ref[...]
    h = jnp.maximum(a * dis_ref[...] + b_ref[...], 0.0)
    o_ref[...] = jnp.dot(h * dis_ref[...], w_ref[...],
                         preferred_element_type=F32)


def _make_final_body(NG, NB, FH):
    def final_body(p0_ref, p1_ref, g_ref, dis_ref, b_ref, batch_ref, wc_ref,
                   bc_ref, o_ref, acc_ref):
        i = pl.program_id(0)

        @pl.when(i == 0)
        def _():
            acc_ref[...] = jnp.zeros_like(acc_ref)

        a = p0_ref[...] + p1_ref[...] + g_ref[...]
        h = jnp.maximum(a * dis_ref[...] + b_ref[...], 0.0)
        b = batch_ref[0, 0, :]
        onehot = (b[None, :] ==
                  lax.broadcasted_iota(jnp.int32, (NG, BT), 0)).astype(F32)
        hx = jnp.concatenate([h, jnp.ones((BT, 1), F32)], axis=1)
        acc_ref[...] += jnp.dot(onehot, hx, preferred_element_type=F32)

        @pl.when(i == NB - 1)
        def _():
            sums = acc_ref[:, :FH]
            cnt = jnp.maximum(acc_ref[:, FH:FH + 1], 1.0)
            pooled = sums / cnt
            logits = jnp.dot(pooled, wc_ref[...], preferred_element_type=F32) \
                + bc_ref[...]
            m = jnp.max(logits, axis=1, keepdims=True)
            lse = jnp.log(jnp.sum(jnp.exp(logits - m), axis=1,
                                  keepdims=True)) + m
            o_ref[...] = logits - lse

    return final_body


def kernel(x, edge_index, batch, W1, b1, W2, b2, W3, b3, Wc, bc):
    N, F_IN = x.shape
    E = edge_index.shape[1]
    NG = 64
    NB = NPAD // BT

    ei = edge_index.astype(jnp.int32)
    xp = jnp.pad(x, ((0, NPAD - N), (0, 0)))
    batchp = jnp.pad(batch.astype(jnp.int32), (0, NPAD - N),
                     constant_values=NG)
    batch3 = batchp.reshape(NB, 1, BT)
    ones_k = jnp.ones((128,), F32)

    degp = _make_deg_kernel(N, E)(ei, ones_k)
    degpT = degp.reshape(NC, NPAD).T

    g1a, g1b, dis = pl.pallas_call(
        _g1_body,
        grid=(NB,),
        in_specs=[pl.BlockSpec((BT, F_IN), lambda i: (i, 0)),
                  pl.BlockSpec((BT, 2), lambda i: (i, 0)),
                  pl.BlockSpec((F_IN, 128), lambda i: (0, 0))],
        out_specs=[pl.BlockSpec((BT, 64), lambda i: (i, 0)),
                   pl.BlockSpec((BT, 64), lambda i: (i, 0)),
                   pl.BlockSpec((BT, 1), lambda i: (i, 0))],
        out_shape=[jax.ShapeDtypeStruct((NPAD, 64), F32),
                   jax.ShapeDtypeStruct((NPAD, 64), F32),
                   jax.ShapeDtypeStruct((NPAD, 1), F32)],
    )(xp, degpT, W1)

    agg1 = _make_agg_split_kernel(N, E, 64)(g1a, g1b, ei)
    g2 = pl.pallas_call(
        _layer12_body,
        grid=(NB,),
        in_specs=[pl.BlockSpec((BT, 64), lambda i: (i, 0)),
                  pl.BlockSpec((BT, 64), lambda i: (NB + i, 0)),
                  pl.BlockSpec((BT, 64), lambda i: (i, 0)),
                  pl.BlockSpec((BT, 64), lambda i: (i, 0)),
                  pl.BlockSpec((BT, 1), lambda i: (i, 0)),
                  pl.BlockSpec((1, 128), lambda i: (0, 0)),
                  pl.BlockSpec((128, 64), lambda i: (0, 0))],
        out_specs=pl.BlockSpec((BT, 64), lambda i: (i, 0)),
        out_shape=jax.ShapeDtypeStruct((NPAD, 64), F32),
    )(agg1, agg1, g1a, g1b, dis, b1.reshape(1, -1), W2)

    agg2 = _make_agg_kernel(N, E, 64)(g2, ei)
    g3 = pl.pallas_call(
        _layer_body,
        grid=(NB,),
        in_specs=[pl.BlockSpec((BT, 64), lambda i: (i, 0)),
                  pl.BlockSpec((BT, 64), lambda i: (NB + i, 0)),
                  pl.BlockSpec((BT, 64), lambda i: (i, 0)),
                  pl.BlockSpec((BT, 1), lambda i: (i, 0)),
                  pl.BlockSpec((1, 64), lambda i: (0, 0)),
                  pl.BlockSpec((64, 32), lambda i: (0, 0))],
        out_specs=pl.BlockSpec((BT, 32), lambda i: (i, 0)),
        out_shape=jax.ShapeDtypeStruct((NPAD, 32), F32),
    )(agg2, agg2, g2, dis, b2.reshape(1, -1), W3)

    agg3 = _make_agg_kernel(N, E, 32)(g3, ei)

    out = pl.pallas_call(
        _make_final_body(NG, NB, 32),
        grid=(NB,),
        in_specs=[pl.BlockSpec((BT, 32), lambda i: (i, 0)),
                  pl.BlockSpec((BT, 32), lambda i: (NB + i, 0)),
                  pl.BlockSpec((BT, 32), lambda i: (i, 0)),
                  pl.BlockSpec((BT, 1), lambda i: (i, 0)),
                  pl.BlockSpec((1, 32), lambda i: (0, 0)),
                  pl.BlockSpec((1, 1, BT), lambda i: (i, 0, 0)),
                  pl.BlockSpec((32, 10), lambda i: (0, 0)),
                  pl.BlockSpec((1, 10), lambda i: (0, 0))],
        out_specs=pl.BlockSpec((NG, 10), lambda i: (0, 0)),
        out_shape=jax.ShapeDtypeStruct((NG, 10), F32),
        scratch_shapes=[pltpu.VMEM((NG, 33), F32)],
    )(agg3, agg3, g3, dis, b3.reshape(1, -1), batch3, Wc, bc.reshape(1, -1))

    return out
